# baseline (speedup 1.0000x reference)
import functools

import jax
import jax.numpy as jnp
from jax import lax
from jax.experimental import pallas as pl
from jax.experimental.pallas import tpu as pltpu
from jax.experimental.pallas import tpu_sc as plsc

B, N, M = 16, 65536, 4096
NC, NS, L = 2, 16, 16
NW = NC * NS
BH, BL = B // 8, 8
MH, ML = M // 128, 128
NH = N // 128
U = BH * MH
UW = BL * ML
PLANE = B * N
OPLANE = B * M


def _make_sc_gather():
    mesh = plsc.VectorSubcoreMesh(core_axis_name="c", subcore_axis_name="s")

    TW = 2 * UW
    Q = 2
    CW = TW // Q
    CE = 3 * CW
    UNROLL = 1

    @functools.partial(
        pl.kernel,
        mesh=mesh,
        compiler_params=pltpu.CompilerParams(
            use_tc_tiling_on_sc=False, needs_layout_passes=False,
            disable_bounds_checks=True, disable_semaphore_checks=True),
        out_type=jax.ShapeDtypeStruct((3 * B * M,), jnp.float32),
        scratch_types=[
            pltpu.VMEM((TW,), jnp.int32),
            pltpu.VMEM((3 * TW,), jnp.int32),
            pltpu.VMEM((3 * TW,), jnp.float32),
        ] + [pltpu.SemaphoreType.DMA] * (2 * Q),
    )
    def sc_gather(xyz_hbm, idx_hbm, out_hbm, idx_v, ent_v, words_v, *sems):
        wid = lax.axis_index("s") * NC + lax.axis_index("c")
        sem_i, sem_g = sems[:Q], sems[Q:]
        u0 = wid * 2
        b_hi = u0 // MH
        a0 = b_hi * (BL * N)
        obase = b_hi * (BL * M) + (u0 % MH) * UW

        cp_i = [
            pltpu.async_copy(
                idx_hbm.at[pl.ds(u0 * UW + q * CW, CW)],
                idx_v.at[pl.ds(q * CW, CW)], sem_i[q])
            for q in range(Q)
        ]

        cp_g = []
        for q in range(Q):
            cp_i[q].wait()

            def build(s, carry, q=q):
                for t in range(UNROLL):
                    j = s * UNROLL + t
                    i = q * (CW // L) + j
                    g = idx_v[pl.ds(i * L, L)]
                    b_lo = lax.rem(i, UW // L) // (ML // L)
                    e0 = (a0 + b_lo * ML) + (
                        lax.shift_right_logical(g, 7) * (BL * ML) + (g & 127))
                    esl = q * CE + j * L
                    ent_v[pl.ds(esl, L)] = e0
                    ent_v[pl.ds(CW + esl, L)] = e0 + PLANE
                    ent_v[pl.ds(2 * CW + esl, L)] = e0 + 2 * PLANE
                return carry

            lax.fori_loop(0, CW // L // UNROLL, build, 0)
            cp_g.append(pltpu.async_copy(
                xyz_hbm.at[ent_v.at[pl.ds(q * CE, CE)]],
                words_v.at[pl.ds(q * CE, CE)], sem_g[q]))

        cp_o = []
        for q in range(Q):
            cp_g[q].wait()
            for c in range(3):
                cp_o.append(pltpu.async_copy(
                    words_v.at[pl.ds(q * CE + c * CW, CW)],
                    out_hbm.at[pl.ds(c * OPLANE + obase + q * CW, CW)],
                    sem_i[q]))
        for cp in cp_o:
            cp.wait()

    return sc_gather


_sc_gather = _make_sc_gather()


def kernel(batch_sample_xyz, input):
    xyz_flat = (
        batch_sample_xyz.transpose(2, 0, 1)
        .reshape(3, BH, BL, NH, 128)
        .transpose(0, 1, 3, 2, 4)
        .reshape(3 * B * N))
    idx_flat = (
        input.astype(jnp.int32)
        .reshape(BH, BL, MH, ML)
        .transpose(0, 2, 1, 3)
        .reshape(B * M))
    out1d = _sc_gather(xyz_flat, idx_flat)
    out = (
        out1d.reshape(3, BH, MH, BL, ML)
        .transpose(1, 3, 2, 4, 0)
        .reshape(B, M, 3))
    return out


# speedup vs baseline: 1.0047x; 1.0047x over previous
import functools

import jax
import jax.numpy as jnp
from jax import lax
from jax.experimental import pallas as pl
from jax.experimental.pallas import tpu as pltpu
from jax.experimental.pallas import tpu_sc as plsc

B, N, M = 16, 65536, 4096
NC, NS, L = 2, 16, 16
NW = NC * NS
BH, BL = B // 8, 8
MH, ML = M // 128, 128
NH = N // 128
U = BH * MH
UW = BL * ML
PLANE = B * N
OPLANE = B * M


def _make_sc_gather():
    mesh = plsc.VectorSubcoreMesh(core_axis_name="c", subcore_axis_name="s")

    TW = 2 * UW
    Q = 2
    CW = TW // Q
    CE = 3 * CW
    UNROLL = 1

    @functools.partial(
        pl.kernel,
        mesh=mesh,
        compiler_params=pltpu.CompilerParams(
            use_tc_tiling_on_sc=False, needs_layout_passes=False,
            skip_device_barrier=True),
        out_type=jax.ShapeDtypeStruct((3 * B * M,), jnp.float32),
        scratch_types=[
            pltpu.VMEM((TW,), jnp.int32),
            pltpu.VMEM((3 * TW,), jnp.int32),
            pltpu.VMEM((3 * TW,), jnp.float32),
        ] + [pltpu.SemaphoreType.DMA] * (2 * Q),
    )
    def sc_gather(xyz_hbm, idx_hbm, out_hbm, idx_v, ent_v, words_v, *sems):
        wid = lax.axis_index("s") * NC + lax.axis_index("c")
        sem_i, sem_g = sems[:Q], sems[Q:]
        u0 = wid * 2
        b_hi = u0 // MH
        a0 = b_hi * (BL * N)
        obase = b_hi * (BL * M) + (u0 % MH) * UW

        cp_i = [
            pltpu.async_copy(
                idx_hbm.at[pl.ds(u0 * UW + q * CW, CW)],
                idx_v.at[pl.ds(q * CW, CW)], sem_i[q])
            for q in range(Q)
        ]

        cp_g = []
        for q in range(Q):
            cp_i[q].wait()

            def build(s, carry, q=q):
                for t in range(UNROLL):
                    j = s * UNROLL + t
                    i = q * (CW // L) + j
                    g = idx_v[pl.ds(i * L, L)]
                    b_lo = lax.rem(i, UW // L) // (ML // L)
                    e0 = (a0 + b_lo * ML) + (
                        lax.shift_right_logical(g, 7) * (BL * ML) + (g & 127))
                    esl = q * CE + j * L
                    ent_v[pl.ds(esl, L)] = e0
                    ent_v[pl.ds(CW + esl, L)] = e0 + PLANE
                    ent_v[pl.ds(2 * CW + esl, L)] = e0 + 2 * PLANE
                return carry

            lax.fori_loop(0, CW // L // UNROLL, build, 0)
            cp_g.append(pltpu.async_copy(
                xyz_hbm.at[ent_v.at[pl.ds(q * CE, CE)]],
                words_v.at[pl.ds(q * CE, CE)], sem_g[q]))

        cp_o = []
        for q in range(Q):
            cp_g[q].wait()
            for c in range(3):
                cp_o.append(pltpu.async_copy(
                    words_v.at[pl.ds(q * CE + c * CW, CW)],
                    out_hbm.at[pl.ds(c * OPLANE + obase + q * CW, CW)],
                    sem_i[q]))
        for cp in cp_o:
            cp.wait()

    return sc_gather


_sc_gather = _make_sc_gather()


def kernel(batch_sample_xyz, input):
    xyz_flat = (
        batch_sample_xyz.transpose(2, 0, 1)
        .reshape(3, BH, BL, NH, 128)
        .transpose(0, 1, 3, 2, 4)
        .reshape(3 * B * N))
    idx_flat = (
        input.astype(jnp.int32)
        .reshape(BH, BL, MH, ML)
        .transpose(0, 2, 1, 3)
        .reshape(B * M))
    out1d = _sc_gather(xyz_flat, idx_flat)
    out = (
        out1d.reshape(3, BH, MH, BL, ML)
        .transpose(1, 3, 2, 4, 0)
        .reshape(B, M, 3))
    return out
